# baseline (speedup 1.0000x reference)
import functools

import jax
import jax.numpy as jnp
from jax import lax
from jax.experimental import pallas as pl
from jax.experimental.pallas import tpu as pltpu
from jax.experimental.pallas import tpu_sc as plsc

B, H, D = 16384, 200, 64
NC, NS = 2, 16
NW = NC * NS
BLK = 128
BLOCKS_PER_W = (B // BLK) // NW
NB = 2
G2 = H // NB

_mesh = plsc.VectorSubcoreMesh(core_axis_name="c", subcore_axis_name="s")


@functools.partial(
    pl.kernel,
    mesh=_mesh,
    out_type=jax.ShapeDtypeStruct((H, 8, B // BLK, 8, BLK), jnp.float32),
    compiler_params=pltpu.CompilerParams(use_tc_tiling_on_sc=False, needs_layout_passes=False),
    scratch_types=[
        pltpu.VMEM((BLK, H), jnp.int32),
        pltpu.VMEM((H, BLK), jnp.int32),
        pltpu.VMEM((BLK, D), jnp.float32),
        pltpu.VMEM((BLK, D), jnp.float32),
        pltpu.VMEM((8, 8, BLK), jnp.float32),
        pltpu.VMEM((8, 8, BLK), jnp.float32),
        pltpu.SemaphoreType.DMA,
        pltpu.SemaphoreType.DMA,
        pltpu.SemaphoreType.DMA,
        pltpu.SemaphoreType.DMA,
        pltpu.SemaphoreType.DMA,
    ],
)
def _gather(idx_hbm, table_hbm, out_hbm,
            xsl, xt, g0, g1, t0, t1, gs0, gs1, os0, os1, xsem):
    wid = lax.axis_index("s") * NC + lax.axis_index("c")
    grows = (g0, g1)
    tbuf = (t0, t1)
    gsem = (gs0, gs1)
    osem = (os0, os1)
    iota = lax.iota(jnp.int32, 16)

    def g_copy(b, h):
        return pltpu.make_async_copy(table_hbm.at[xt.at[h]], grows[b],
                                     gsem[b])

    def o_copy(b, h, bjg):
        return pltpu.make_async_copy(tbuf[b], out_hbm.at[h, :, bjg],
                                     osem[b])

    def transpose_chunk(b):
        for f in range(D):
            col = jnp.full((16,), f, jnp.int32)
            for g in range(8):
                v = plsc.load_gather(grows[b], [iota + g * 16, col])
                tbuf[b][f // 8, f % 8, pl.ds(g * 16, 16)] = v

    def block_body(k, carry):
        bjg = wid * BLOCKS_PER_W + k

        pltpu.make_async_copy(
            idx_hbm.at[pl.ds(bjg * BLK, BLK)], xsl, xsem).start()
        pltpu.make_async_copy(
            idx_hbm.at[pl.ds(bjg * BLK, BLK)], xsl, xsem).wait()

        def xtr(h, c):
            hv = jnp.full((16,), 0, jnp.int32) + h
            for g in range(8):
                v = plsc.load_gather(xsl, [iota + g * 16, hv])
                xt[h, pl.ds(g * 16, 16)] = v
            return c
        lax.fori_loop(0, H, xtr, 0)

        for m in range(NB):
            g_copy(m, m).start()

        def hstep(gg, c):
            for b in range(NB):
                h = gg * NB + b
                g_copy(b, h).wait()

                @pl.when(gg > 0)
                def _():
                    o_copy(b, h - NB, bjg).wait()

                transpose_chunk(b)
                o_copy(b, h, bjg).start()

                @pl.when(gg < G2 - 1)
                def _():
                    g_copy(b, h + NB).start()
            return c

        lax.fori_loop(0, G2, hstep, 0)
        o_copy(0, H - 2, bjg).wait()
        o_copy(1, H - 1, bjg).wait()
        return carry

    lax.fori_loop(0, BLOCKS_PER_W, block_body, 0)


def kernel(x, emb):
    out5 = _gather(x.astype(jnp.int32), emb)
    return out5.transpose(2, 4, 0, 1, 3).reshape(B, H, D)


# speedup vs baseline: 1.9210x; 1.9210x over previous
import functools

import jax
import jax.numpy as jnp
from jax import lax
from jax.experimental import pallas as pl
from jax.experimental.pallas import tpu as pltpu
from jax.experimental.pallas import tpu_sc as plsc

B, H, D = 16384, 200, 64
NC, NS = 2, 16
NW = NC * NS
BLK = 128
BLOCKS_PER_W = (B // BLK) // NW
NB = 2
G2 = H // NB

_mesh = plsc.VectorSubcoreMesh(core_axis_name="c", subcore_axis_name="s")


@functools.partial(
    pl.kernel,
    mesh=_mesh,
    out_type=jax.ShapeDtypeStruct((H, 8, B // BLK, 8, BLK), jnp.float32),
    compiler_params=pltpu.CompilerParams(use_tc_tiling_on_sc=False, needs_layout_passes=False),
    scratch_types=[
        pltpu.VMEM((BLK, H), jnp.int32),
        pltpu.VMEM((H, BLK), jnp.int32),
        pltpu.VMEM((BLK, D), jnp.float32),
        pltpu.VMEM((BLK, D), jnp.float32),
        pltpu.VMEM((8, 8, BLK), jnp.float32),
        pltpu.VMEM((8, 8, BLK), jnp.float32),
        pltpu.SemaphoreType.DMA,
        pltpu.SemaphoreType.DMA,
        pltpu.SemaphoreType.DMA,
        pltpu.SemaphoreType.DMA,
        pltpu.SemaphoreType.DMA,
    ],
)
def _gather(idx_hbm, table_hbm, out_hbm,
            xsl, xt, g0, g1, t0, t1, gs0, gs1, os0, os1, xsem):
    wid = lax.axis_index("s") * NC + lax.axis_index("c")
    grows = (g0, g1)
    tbuf = (t0, t1)
    gsem = (gs0, gs1)
    osem = (os0, os1)
    iota = lax.iota(jnp.int32, 16)

    def g_copy(b, h):
        return pltpu.make_async_copy(table_hbm.at[xt.at[h]], grows[b],
                                     gsem[b])

    def o_copy(b, h, bjg):
        return pltpu.make_async_copy(tbuf[b], out_hbm.at[h, :, bjg],
                                     osem[b])

    def transpose_chunk(b):
        @plsc.parallel_loop(0, D, unroll=8)
        def _(f):
            col = jnp.full((16,), 0, jnp.int32) + f
            for g in range(8):
                v = plsc.load_gather(grows[b], [iota + g * 16, col])
                tbuf[b][f // 8, f % 8, pl.ds(g * 16, 16)] = v

    def block_body(k, carry):
        bjg = wid * BLOCKS_PER_W + k

        pltpu.make_async_copy(
            idx_hbm.at[pl.ds(bjg * BLK, BLK)], xsl, xsem).start()
        pltpu.make_async_copy(
            idx_hbm.at[pl.ds(bjg * BLK, BLK)], xsl, xsem).wait()

        @plsc.parallel_loop(0, H, unroll=8)
        def _(h):
            hv = jnp.full((16,), 0, jnp.int32) + h
            for g in range(8):
                v = plsc.load_gather(xsl, [iota + g * 16, hv])
                xt[h, pl.ds(g * 16, 16)] = v

        for m in range(NB):
            g_copy(m, m).start()

        def hstep(gg, c):
            for b in range(NB):
                h = gg * NB + b
                g_copy(b, h).wait()

                @pl.when(gg > 0)
                def _():
                    o_copy(b, h - NB, bjg).wait()

                transpose_chunk(b)
                o_copy(b, h, bjg).start()

                @pl.when(gg < G2 - 1)
                def _():
                    g_copy(b, h + NB).start()
            return c

        lax.fori_loop(0, G2, hstep, 0)
        o_copy(0, H - 2, bjg).wait()
        o_copy(1, H - 1, bjg).wait()
        return carry

    lax.fori_loop(0, BLOCKS_PER_W, block_body, 0)


def kernel(x, emb):
    out5 = _gather(x.astype(jnp.int32), emb)
    return out5.transpose(2, 4, 0, 1, 3).reshape(B, H, D)


# speedup vs baseline: 2.6977x; 1.4043x over previous
import functools

import jax
import jax.numpy as jnp
from jax import lax
from jax.experimental import pallas as pl
from jax.experimental.pallas import tpu as pltpu
from jax.experimental.pallas import tpu_sc as plsc

B, H, D = 16384, 200, 64
NC, NS = 2, 16
NW = NC * NS
BLK = 128
BLOCKS_PER_W = (B // BLK) // NW
NB = 2
G2 = H // NB

_mesh = plsc.VectorSubcoreMesh(core_axis_name="c", subcore_axis_name="s")


@functools.partial(
    pl.kernel,
    mesh=_mesh,
    out_type=jax.ShapeDtypeStruct((H, 8, B // BLK, 8, BLK), jnp.float32),
    compiler_params=pltpu.CompilerParams(use_tc_tiling_on_sc=False, needs_layout_passes=False),
    scratch_types=[
        pltpu.VMEM((BLK, H), jnp.int32),
        pltpu.VMEM((H, BLK), jnp.int32),
        pltpu.VMEM((BLK, D), jnp.float32),
        pltpu.VMEM((BLK, D), jnp.float32),
        pltpu.VMEM((D, BLK), jnp.float32),
        pltpu.VMEM((D, BLK), jnp.float32),
        pltpu.SemaphoreType.DMA,
        pltpu.SemaphoreType.DMA,
        pltpu.SemaphoreType.DMA,
        pltpu.SemaphoreType.DMA,
        pltpu.SemaphoreType.DMA,
    ],
)
def _gather(idx_hbm, table_hbm, out_hbm,
            xsl, xt, g0, g1, t0, t1, gs0, gs1, os0, os1, xsem):
    wid = lax.axis_index("s") * NC + lax.axis_index("c")
    grows = (g0, g1)
    tbuf = (t0, t1)
    gsem = (gs0, gs1)
    osem = (os0, os1)
    iota = lax.iota(jnp.int32, 16)

    def g_copy(b, h):
        return pltpu.make_async_copy(table_hbm.at[xt.at[h]], grows[b],
                                     gsem[b])

    def o_copy(b, h, bjg, fi):
        return pltpu.make_async_copy(tbuf[b].at[pl.ds(fi * 8, 8)],
                                     out_hbm.at[h, fi, bjg], osem[b])

    def o_start(b, h, bjg):
        for fi in range(8):
            o_copy(b, h, bjg, fi).start()

    def o_wait(b, h, bjg):
        for fi in range(8):
            o_copy(b, h, bjg, fi).wait()

    def transpose_chunk(b):
        @plsc.parallel_loop(0, BLK, step=16, unroll=1)
        def _(l0):
            li = iota + l0
            for fi0 in range(0, D, 16):
                for r in range(16):
                    fv = (iota ^ r) + fi0
                    v = plsc.load_gather(grows[b], [li, fv])
                    plsc.store_scatter(tbuf[b], [fv, li], v)

    def block_body(k, carry):
        bjg = wid * BLOCKS_PER_W + k

        pltpu.make_async_copy(
            idx_hbm.at[pl.ds(bjg * BLK, BLK)], xsl, xsem).start()
        pltpu.make_async_copy(
            idx_hbm.at[pl.ds(bjg * BLK, BLK)], xsl, xsem).wait()

        @plsc.parallel_loop(0, H, unroll=8)
        def _(h):
            hv = jnp.full((16,), 0, jnp.int32) + h
            for g in range(8):
                v = plsc.load_gather(xsl, [iota + g * 16, hv])
                xt[h, pl.ds(g * 16, 16)] = v

        for m in range(NB):
            g_copy(m, m).start()

        def hstep(gg, c):
            for b in range(NB):
                h = gg * NB + b
                g_copy(b, h).wait()

                @pl.when(gg > 0)
                def _():
                    o_wait(b, h - NB, bjg)

                transpose_chunk(b)
                o_start(b, h, bjg)

                @pl.when(gg < G2 - 1)
                def _():
                    g_copy(b, h + NB).start()
            return c

        lax.fori_loop(0, G2, hstep, 0)
        o_wait(0, H - 2, bjg)
        o_wait(1, H - 1, bjg)
        return carry

    lax.fori_loop(0, BLOCKS_PER_W, block_body, 0)


def kernel(x, emb):
    out5 = _gather(x.astype(jnp.int32), emb)
    return out5.transpose(2, 4, 0, 1, 3).reshape(B, H, D)


# speedup vs baseline: 3.1465x; 1.1664x over previous
import functools

import jax
import jax.numpy as jnp
from jax import lax
from jax.experimental import pallas as pl
from jax.experimental.pallas import tpu as pltpu
from jax.experimental.pallas import tpu_sc as plsc

B, H, D = 16384, 200, 64
NC, NS = 2, 16
NW = NC * NS
BLK = 128
BLOCKS_PER_W = (B // BLK) // NW
NB = 2
G2 = H // NB

_mesh = plsc.VectorSubcoreMesh(core_axis_name="c", subcore_axis_name="s")


@functools.partial(
    pl.kernel,
    mesh=_mesh,
    out_type=jax.ShapeDtypeStruct((H, 8, B // BLK, 8, BLK), jnp.float32),
    compiler_params=pltpu.CompilerParams(use_tc_tiling_on_sc=False, needs_layout_passes=False),
    scratch_types=[
        pltpu.VMEM((BLK, H), jnp.int32),
        pltpu.VMEM((H, BLK), jnp.int32),
        pltpu.VMEM((BLK, D), jnp.float32),
        pltpu.VMEM((BLK, D), jnp.float32),
        pltpu.VMEM((D, BLK), jnp.float32),
        pltpu.VMEM((D, BLK), jnp.float32),
        pltpu.SemaphoreType.DMA,
        pltpu.SemaphoreType.DMA,
        pltpu.SemaphoreType.DMA,
        pltpu.SemaphoreType.DMA,
        pltpu.SemaphoreType.DMA,
    ],
)
def _gather(idx_hbm, table_hbm, out_hbm,
            xsl, xt, g0, g1, t0, t1, gs0, gs1, os0, os1, xsem):
    wid = lax.axis_index("s") * NC + lax.axis_index("c")
    grows = (g0, g1)
    tbuf = (t0, t1)
    gsem = (gs0, gs1)
    osem = (os0, os1)
    iota = lax.iota(jnp.int32, 16)

    def g_copy(b, h):
        return pltpu.make_async_copy(table_hbm.at[xt.at[h]], grows[b],
                                     gsem[b])

    def o_copy(b, h, bjg, fi):
        return pltpu.make_async_copy(tbuf[b].at[pl.ds(fi * 8, 8)],
                                     out_hbm.at[h, fi, bjg], osem[b])

    def o_start(b, h, bjg):
        for fi in range(8):
            o_copy(b, h, bjg, fi).start()

    def o_wait(b, h, bjg):
        for fi in range(8):
            o_copy(b, h, bjg, fi).wait()

    def transpose_chunk(b):
        @plsc.parallel_loop(0, BLK, step=16, unroll=2)
        def _(l0):
            li = iota + l0
            for fi0 in range(0, D, 16):
                for r in range(16):
                    fv = (iota ^ r) + fi0
                    v = plsc.load_gather(grows[b], [li, fv])
                    plsc.store_scatter(tbuf[b], [fv, li], v)

    def block_body(k, carry):
        bjg = wid * BLOCKS_PER_W + k

        pltpu.make_async_copy(
            idx_hbm.at[pl.ds(bjg * BLK, BLK)], xsl, xsem).start()
        pltpu.make_async_copy(
            idx_hbm.at[pl.ds(bjg * BLK, BLK)], xsl, xsem).wait()

        @plsc.parallel_loop(0, H, unroll=8)
        def _(h):
            hv = jnp.full((16,), 0, jnp.int32) + h
            for g in range(8):
                v = plsc.load_gather(xsl, [iota + g * 16, hv])
                xt[h, pl.ds(g * 16, 16)] = v

        for m in range(NB):
            g_copy(m, m).start()

        def hstep(gg, c):
            for b in range(NB):
                h = gg * NB + b
                g_copy(b, h).wait()

                @pl.when(gg > 0)
                def _():
                    o_wait(b, h - NB, bjg)

                transpose_chunk(b)
                o_start(b, h, bjg)

                @pl.when(gg < G2 - 1)
                def _():
                    g_copy(b, h + NB).start()
            return c

        lax.fori_loop(0, G2, hstep, 0)
        o_wait(0, H - 2, bjg)
        o_wait(1, H - 1, bjg)
        return carry

    lax.fori_loop(0, BLOCKS_PER_W, block_body, 0)


def kernel(x, emb):
    out5 = _gather(x.astype(jnp.int32), emb)
    return out5.transpose(2, 4, 0, 1, 3).reshape(B, H, D)


# speedup vs baseline: 4.5569x; 1.4482x over previous
import functools

import jax
import jax.numpy as jnp
from jax import lax
from jax.experimental import pallas as pl
from jax.experimental.pallas import tpu as pltpu
from jax.experimental.pallas import tpu_sc as plsc

B, H, D = 16384, 200, 64
NC, NS = 2, 16
NW = NC * NS
BLK = 128
BLOCKS_PER_W = (B // BLK) // NW
NB = 2
G2 = H // NB

_mesh = plsc.VectorSubcoreMesh(core_axis_name="c", subcore_axis_name="s")


@functools.partial(
    pl.kernel,
    mesh=_mesh,
    out_type=jax.ShapeDtypeStruct((H, 8, B // BLK, 8, BLK), jnp.float32),
    compiler_params=pltpu.CompilerParams(use_tc_tiling_on_sc=False, needs_layout_passes=False),
    scratch_types=[
        pltpu.VMEM((BLK, H), jnp.int32),
        pltpu.VMEM((H, BLK), jnp.int32),
        pltpu.VMEM((BLK, D), jnp.float32),
        pltpu.VMEM((BLK, D), jnp.float32),
        pltpu.VMEM((D, BLK), jnp.float32),
        pltpu.VMEM((D, BLK), jnp.float32),
        pltpu.SemaphoreType.DMA,
        pltpu.SemaphoreType.DMA,
        pltpu.SemaphoreType.DMA,
        pltpu.SemaphoreType.DMA,
        pltpu.SemaphoreType.DMA,
    ],
)
def _gather(idx_hbm, table_hbm, out_hbm,
            xsl, xt, g0, g1, t0, t1, gs0, gs1, os0, os1, xsem):
    wid = lax.axis_index("s") * NC + lax.axis_index("c")
    grows = (g0, g1)
    tbuf = (t0, t1)
    gsem = (gs0, gs1)
    osem = (os0, os1)
    iota = lax.iota(jnp.int32, 16)

    def g_copy(b, h):
        return pltpu.make_async_copy(table_hbm.at[xt.at[h]], grows[b],
                                     gsem[b])

    def o_copy(b, h, bjg, fi):
        return pltpu.make_async_copy(tbuf[b].at[pl.ds(fi * 8, 8)],
                                     out_hbm.at[h, fi, bjg], osem[b])

    def o_start(b, h, bjg):
        for fi in range(8):
            o_copy(b, h, bjg, fi).start()

    def o_wait(b, h, bjg):
        for fi in range(8):
            o_copy(b, h, bjg, fi).wait()

    def transpose_chunk(b):
        @plsc.parallel_loop(0, BLK, step=16, unroll=4)
        def _(l0):
            li = iota + l0
            for fi0 in range(0, D, 16):
                for r in range(16):
                    fv = (iota ^ r) + fi0
                    v = plsc.load_gather(grows[b], [li, fv])
                    plsc.store_scatter(tbuf[b], [fv, li], v)

    def block_body(k, carry):
        bjg = wid * BLOCKS_PER_W + k

        pltpu.make_async_copy(
            idx_hbm.at[pl.ds(bjg * BLK, BLK)], xsl, xsem).start()
        pltpu.make_async_copy(
            idx_hbm.at[pl.ds(bjg * BLK, BLK)], xsl, xsem).wait()

        @plsc.parallel_loop(0, H, unroll=8)
        def _(h):
            hv = jnp.full((16,), 0, jnp.int32) + h
            for g in range(8):
                v = plsc.load_gather(xsl, [iota + g * 16, hv])
                xt[h, pl.ds(g * 16, 16)] = v

        for m in range(NB):
            g_copy(m, m).start()

        def hstep(gg, c):
            for b in range(NB):
                h = gg * NB + b
                g_copy(b, h).wait()

                @pl.when(gg > 0)
                def _():
                    o_wait(b, h - NB, bjg)

                transpose_chunk(b)
                o_start(b, h, bjg)

                @pl.when(gg < G2 - 1)
                def _():
                    g_copy(b, h + NB).start()
            return c

        lax.fori_loop(0, G2, hstep, 0)
        o_wait(0, H - 2, bjg)
        o_wait(1, H - 1, bjg)
        return carry

    lax.fori_loop(0, BLOCKS_PER_W, block_body, 0)


def kernel(x, emb):
    out5 = _gather(x.astype(jnp.int32), emb)
    return out5.transpose(2, 4, 0, 1, 3).reshape(B, H, D)
